# Initial kernel scaffold; baseline (speedup 1.0000x reference)
import jax
import jax.numpy as jnp
from jax.experimental import pallas as pl


def kernel(x, ei_supplies, ei_competes, Wl0s, bl0s, Wr0s, Wl0c, bl0c, Wr0c, Wl1s, bl1s, Wr1s, Wl1c, bl1c, Wr1c, linW, linb):
    raise NotImplementedError("write your pallas kernel here")


# speedup vs baseline: 2.4383x; 2.4383x over previous
import jax
import jax.numpy as jnp
from jax import lax
from jax.experimental import pallas as pl
from jax.experimental.pallas import tpu as pltpu
from jax.experimental.pallas import tpu_sc as plsc

N = 10000
D = 128
E = 320000
NC = 2
NS = 16
K = 64
J = 2
NG = 160
W = NG * J * K
EPAD = W * NS
NACC = 10112
RPT = NACC // NS
RB = 1000
_CHUNKS = [(i * K, K) for i in range(RPT // K)] + [(RPT - RPT % K, RPT % K)]


def _sc_agg_body(ei, table, zrow_h, acc_out,
                 srcidx, dstidx, rows, zrow, acc, isem, gsem, ssem):
    c = lax.axis_index("c")
    s = lax.axis_index("s")
    base = s * RPT

    pltpu.sync_copy(zrow_h, zrow)
    for off, sz in _CHUNKS:
        pltpu.sync_copy(zrow.at[pl.ds(0, sz)], acc.at[pl.ds(base + off, sz)])
    plsc.subcore_barrier()

    def _group(g, carry):
        fi_src = ((c * 2 + 0) * NS + s) * NG + g
        fi_dst = ((c * 2 + 1) * NS + s) * NG + g
        dsrc = pltpu.async_copy(ei.at[fi_src], srcidx, isem)
        ddst = pltpu.async_copy(ei.at[fi_dst], dstidx, isem)
        dsrc.wait()
        ddst.wait()
        gds = [pltpu.async_copy(table.at[srcidx.at[j]], rows.at[j], gsem)
               for j in range(J)]
        for dsc in gds:
            dsc.wait()
        sds = [pltpu.async_copy(rows.at[j], acc.at[dstidx.at[j]], ssem,
                                add=True) for j in range(J)]
        for dsc in sds:
            dsc.wait()
        return carry

    lax.fori_loop(0, NG, _group, 0)
    plsc.subcore_barrier()

    out_base = c * NACC + base
    for off, sz in _CHUNKS:
        pltpu.sync_copy(acc.at[pl.ds(base + off, sz)], zrow.at[pl.ds(0, sz)])
        pltpu.sync_copy(zrow.at[pl.ds(0, sz)],
                        acc_out.at[pl.ds(out_base + off, sz)])


_sc_agg = pl.kernel(
    _sc_agg_body,
    out_type=jax.ShapeDtypeStruct((NC * NACC, D), jnp.float32),
    mesh=plsc.VectorSubcoreMesh(core_axis_name="c", subcore_axis_name="s",
                                num_cores=NC, num_subcores=NS),
    scratch_types=[
        pltpu.VMEM((J, K), jnp.int32),
        pltpu.VMEM((J, K), jnp.int32),
        pltpu.VMEM((J, K, D), jnp.float32),
        pltpu.VMEM((K, D), jnp.float32),
        pltpu.VMEM_SHARED((NACC, D), jnp.float32),
        pltpu.SemaphoreType.DMA,
        pltpu.SemaphoreType.DMA,
        pltpu.SemaphoreType.DMA,
    ],
)


def _tc0_body(acc_ref, cnt_ref, x_ref, wls_ref, wlc_ref, wrs_ref, wrc_ref,
              bls_ref, blc_ref, h_ref):
    inv_s = 1.0 / jnp.maximum(cnt_ref[:, 0:1], 1.0)
    inv_c = 1.0 / jnp.maximum(cnt_ref[:, 1:2], 1.0)
    r = jnp.dot(acc_ref[0] * inv_s, wls_ref[...],
                preferred_element_type=jnp.float32)
    r = r + jnp.dot(acc_ref[1] * inv_c, wlc_ref[...],
                    preferred_element_type=jnp.float32)
    r = r + jnp.dot(x_ref[...], wrs_ref[...] + wrc_ref[...],
                    preferred_element_type=jnp.float32)
    r = r + bls_ref[...] + blc_ref[...]
    h_ref[...] = jnp.maximum(r, 0.0)


def _tc1_body(acc_ref, cnt_ref, h_ref, wls_ref, wlc_ref, wrs_ref, wrc_ref,
              bls_ref, blc_ref, lw_ref, lb_ref, o_ref):
    inv_s = 1.0 / jnp.maximum(cnt_ref[:, 0:1], 1.0)
    inv_c = 1.0 / jnp.maximum(cnt_ref[:, 1:2], 1.0)
    g = jnp.dot(acc_ref[0] * inv_s, wls_ref[...],
                preferred_element_type=jnp.float32)
    g = g + jnp.dot(acc_ref[1] * inv_c, wlc_ref[...],
                    preferred_element_type=jnp.float32)
    g = g + jnp.dot(h_ref[...], wrs_ref[...] + wrc_ref[...],
                    preferred_element_type=jnp.float32)
    g = jnp.maximum(g + bls_ref[...] + blc_ref[...], 0.0)
    o_ref[...] = jnp.dot(g, lw_ref[...],
                         preferred_element_type=jnp.float32) + lb_ref[...]


def _tc0(acc, cnt, x, wls, wlc, wrs, wrc, bls, blc):
    wspec = pl.BlockSpec((D, D), lambda i: (0, 0))
    bspec = pl.BlockSpec((1, D), lambda i: (0, 0))
    return pl.pallas_call(
        _tc0_body,
        grid=(N // RB,),
        in_specs=[
            pl.BlockSpec((NC, RB, D), lambda i: (0, i, 0)),
            pl.BlockSpec((RB, NC), lambda i: (i, 0)),
            pl.BlockSpec((RB, D), lambda i: (i, 0)),
            wspec, wspec, wspec, wspec, bspec, bspec,
        ],
        out_specs=pl.BlockSpec((RB, D), lambda i: (i, 0)),
        out_shape=jax.ShapeDtypeStruct((N, D), jnp.float32),
    )(acc, cnt, x, wls, wlc, wrs, wrc, bls, blc)


def _tc1(acc, cnt, h, wls, wlc, wrs, wrc, bls, blc, lw, lb):
    wspec = pl.BlockSpec((D, D), lambda i: (0, 0))
    bspec = pl.BlockSpec((1, D), lambda i: (0, 0))
    return pl.pallas_call(
        _tc1_body,
        grid=(N // RB,),
        in_specs=[
            pl.BlockSpec((NC, RB, D), lambda i: (0, i, 0)),
            pl.BlockSpec((RB, NC), lambda i: (i, 0)),
            pl.BlockSpec((RB, D), lambda i: (i, 0)),
            wspec, wspec, wspec, wspec, bspec, bspec, wspec, bspec,
        ],
        out_specs=pl.BlockSpec((RB, D), lambda i: (i, 0)),
        out_shape=jax.ShapeDtypeStruct((N, D), jnp.float32),
    )(acc, cnt, h, wls, wlc, wrs, wrc, bls, blc, lw, lb)


def _pack_edges(ei_s, ei_c):
    pad = EPAD - E

    def prep(ei):
        src = jnp.concatenate([ei[0], jnp.zeros((pad,), jnp.int32)])
        dst = jnp.concatenate([ei[1], jnp.full((pad,), N, jnp.int32)])
        return jnp.stack([src, dst])

    packed = jnp.stack([prep(ei_s), prep(ei_c)])
    return packed.reshape(NC * 2 * NS * NG, J, K)


def kernel(x, ei_supplies, ei_competes, Wl0s, bl0s, Wr0s, Wl0c, bl0c, Wr0c,
           Wl1s, bl1s, Wr1s, Wl1c, bl1c, Wr1c, linW, linb):
    ei = _pack_edges(ei_supplies, ei_competes)
    zrow_h = jnp.zeros((K, D), jnp.float32)

    ones_e = jnp.ones((E,), jnp.float32)
    cnt = jnp.stack(
        [jax.ops.segment_sum(ones_e, ei_supplies[1], num_segments=N),
         jax.ops.segment_sum(ones_e, ei_competes[1], num_segments=N)],
        axis=1)

    acc_a = _sc_agg(ei, x, zrow_h).reshape(NC, NACC, D)
    h = _tc0(acc_a, cnt, x, Wl0s, Wl0c, Wr0s, Wr0c,
             bl0s.reshape(1, D), bl0c.reshape(1, D))
    acc_b = _sc_agg(ei, h, zrow_h).reshape(NC, NACC, D)
    return _tc1(acc_b, cnt, h, Wl1s, Wl1c, Wr1s, Wr1c,
                bl1s.reshape(1, D), bl1c.reshape(1, D),
                linW, linb.reshape(1, D))


# speedup vs baseline: 2.9277x; 1.2007x over previous
import jax
import jax.numpy as jnp
from jax import lax
from jax.experimental import pallas as pl
from jax.experimental.pallas import tpu as pltpu
from jax.experimental.pallas import tpu_sc as plsc

N = 10000
D = 128
E = 320000
NC = 2
NS = 16
K = 64
J = 2
NG = 160
W = NG * J * K
EPAD = W * NS
NACC = 10112
RPT = NACC // NS
RB = 1000
_CHUNKS = [(i * K, K) for i in range(RPT // K)] + [(RPT - RPT % K, RPT % K)]


def _sc_agg_body(ei, table, zrow_h, acc_out,
                 srcidx, dstidx, rows, zrow, acc, isem, gsem, ssem):
    c = lax.axis_index("c")
    s = lax.axis_index("s")
    base = s * RPT

    pltpu.sync_copy(zrow_h, zrow)
    for off, sz in _CHUNKS:
        pltpu.sync_copy(zrow.at[pl.ds(0, sz)], acc.at[pl.ds(base + off, sz)])
    plsc.subcore_barrier()

    def _group(g, carry):
        fi_src = ((c * 2 + 0) * NS + s) * NG + g
        fi_dst = ((c * 2 + 1) * NS + s) * NG + g
        dsrc = pltpu.async_copy(ei.at[fi_src], srcidx, isem)
        ddst = pltpu.async_copy(ei.at[fi_dst], dstidx, isem)
        dsrc.wait()
        ddst.wait()
        gds = [pltpu.async_copy(table.at[srcidx.at[j]], rows.at[j], gsem)
               for j in range(J)]
        for dsc in gds:
            dsc.wait()
        sds = [pltpu.async_copy(rows.at[j], acc.at[dstidx.at[j]], ssem,
                                add=True) for j in range(J)]
        for dsc in sds:
            dsc.wait()
        return carry

    lax.fori_loop(0, NG, _group, 0)
    plsc.subcore_barrier()

    out_base = c * NACC + base
    for off, sz in _CHUNKS:
        pltpu.sync_copy(acc.at[pl.ds(base + off, sz)], zrow.at[pl.ds(0, sz)])
        pltpu.sync_copy(zrow.at[pl.ds(0, sz)],
                        acc_out.at[pl.ds(out_base + off, sz)])


_sc_agg = pl.kernel(
    _sc_agg_body,
    out_type=jax.ShapeDtypeStruct((NC * NACC, D), jnp.float32),
    mesh=plsc.VectorSubcoreMesh(core_axis_name="c", subcore_axis_name="s",
                                num_cores=NC, num_subcores=NS),
    scratch_types=[
        pltpu.VMEM((J, K), jnp.int32),
        pltpu.VMEM((J, K), jnp.int32),
        pltpu.VMEM((J, K, D), jnp.float32),
        pltpu.VMEM((K, D), jnp.float32),
        pltpu.VMEM_SHARED((NACC, D), jnp.float32),
        pltpu.SemaphoreType.DMA,
        pltpu.SemaphoreType.DMA,
        pltpu.SemaphoreType.DMA,
    ],
)


def _sc_cnt_body(ei, zrow_h, ones_h, cnt_out,
                 dstidx, onesb, zrow, cnt, isem, ssem):
    c = lax.axis_index("c")
    s = lax.axis_index("s")
    base = s * RPT

    pltpu.sync_copy(zrow_h, zrow)
    pltpu.sync_copy(ones_h, onesb)
    for off, sz in _CHUNKS:
        pltpu.sync_copy(zrow.at[pl.ds(0, sz)], cnt.at[pl.ds(base + off, sz)])
    plsc.subcore_barrier()

    def _group(g, carry):
        fi_dst = ((c * 2 + 1) * NS + s) * NG + g
        pltpu.async_copy(ei.at[fi_dst], dstidx, isem).wait()
        sds = [pltpu.async_copy(onesb, cnt.at[dstidx.at[j]], ssem,
                                add=True) for j in range(J)]
        for dsc in sds:
            dsc.wait()
        return carry

    lax.fori_loop(0, NG, _group, 0)
    plsc.subcore_barrier()

    out_base = c * NACC + base
    for off, sz in _CHUNKS:
        pltpu.sync_copy(cnt.at[pl.ds(base + off, sz)], zrow.at[pl.ds(0, sz)])
        pltpu.sync_copy(zrow.at[pl.ds(0, sz)],
                        cnt_out.at[pl.ds(out_base + off, sz)])


_sc_cnt = pl.kernel(
    _sc_cnt_body,
    out_type=jax.ShapeDtypeStruct((NC * NACC, D), jnp.float32),
    mesh=plsc.VectorSubcoreMesh(core_axis_name="c", subcore_axis_name="s",
                                num_cores=NC, num_subcores=NS),
    scratch_types=[
        pltpu.VMEM((J, K), jnp.int32),
        pltpu.VMEM((K, D), jnp.float32),
        pltpu.VMEM((K, D), jnp.float32),
        pltpu.VMEM_SHARED((NACC, D), jnp.float32),
        pltpu.SemaphoreType.DMA,
        pltpu.SemaphoreType.DMA,
    ],
)


def _tc0_body(acc_ref, cnt_ref, x_ref, wls_ref, wlc_ref, wrs_ref, wrc_ref,
              bls_ref, blc_ref, h_ref):
    inv_s = 1.0 / jnp.maximum(cnt_ref[:, 0:1], 1.0)
    inv_c = 1.0 / jnp.maximum(cnt_ref[:, 1:2], 1.0)
    r = jnp.dot(acc_ref[0] * inv_s, wls_ref[...],
                preferred_element_type=jnp.float32)
    r = r + jnp.dot(acc_ref[1] * inv_c, wlc_ref[...],
                    preferred_element_type=jnp.float32)
    r = r + jnp.dot(x_ref[...], wrs_ref[...] + wrc_ref[...],
                    preferred_element_type=jnp.float32)
    r = r + bls_ref[...] + blc_ref[...]
    h_ref[...] = jnp.maximum(r, 0.0)


def _tc1_body(acc_ref, cnt_ref, h_ref, wls_ref, wlc_ref, wrs_ref, wrc_ref,
              bls_ref, blc_ref, lw_ref, lb_ref, o_ref):
    inv_s = 1.0 / jnp.maximum(cnt_ref[:, 0:1], 1.0)
    inv_c = 1.0 / jnp.maximum(cnt_ref[:, 1:2], 1.0)
    g = jnp.dot(acc_ref[0] * inv_s, wls_ref[...],
                preferred_element_type=jnp.float32)
    g = g + jnp.dot(acc_ref[1] * inv_c, wlc_ref[...],
                    preferred_element_type=jnp.float32)
    g = g + jnp.dot(h_ref[...], wrs_ref[...] + wrc_ref[...],
                    preferred_element_type=jnp.float32)
    g = jnp.maximum(g + bls_ref[...] + blc_ref[...], 0.0)
    o_ref[...] = jnp.dot(g, lw_ref[...],
                         preferred_element_type=jnp.float32) + lb_ref[...]


def _tc0(acc, cnt, x, wls, wlc, wrs, wrc, bls, blc):
    wspec = pl.BlockSpec((D, D), lambda i: (0, 0))
    bspec = pl.BlockSpec((1, D), lambda i: (0, 0))
    return pl.pallas_call(
        _tc0_body,
        grid=(N // RB,),
        in_specs=[
            pl.BlockSpec((NC, RB, D), lambda i: (0, i, 0)),
            pl.BlockSpec((RB, NC), lambda i: (i, 0)),
            pl.BlockSpec((RB, D), lambda i: (i, 0)),
            wspec, wspec, wspec, wspec, bspec, bspec,
        ],
        out_specs=pl.BlockSpec((RB, D), lambda i: (i, 0)),
        out_shape=jax.ShapeDtypeStruct((N, D), jnp.float32),
    )(acc, cnt, x, wls, wlc, wrs, wrc, bls, blc)


def _tc1(acc, cnt, h, wls, wlc, wrs, wrc, bls, blc, lw, lb):
    wspec = pl.BlockSpec((D, D), lambda i: (0, 0))
    bspec = pl.BlockSpec((1, D), lambda i: (0, 0))
    return pl.pallas_call(
        _tc1_body,
        grid=(N // RB,),
        in_specs=[
            pl.BlockSpec((NC, RB, D), lambda i: (0, i, 0)),
            pl.BlockSpec((RB, NC), lambda i: (i, 0)),
            pl.BlockSpec((RB, D), lambda i: (i, 0)),
            wspec, wspec, wspec, wspec, bspec, bspec, wspec, bspec,
        ],
        out_specs=pl.BlockSpec((RB, D), lambda i: (i, 0)),
        out_shape=jax.ShapeDtypeStruct((N, D), jnp.float32),
    )(acc, cnt, h, wls, wlc, wrs, wrc, bls, blc, lw, lb)


def _pack_edges(ei_s, ei_c):
    pad = EPAD - E

    def prep(ei):
        src = jnp.concatenate([ei[0], jnp.zeros((pad,), jnp.int32)])
        dst = jnp.concatenate([ei[1], jnp.full((pad,), N, jnp.int32)])
        return jnp.stack([src, dst])

    packed = jnp.stack([prep(ei_s), prep(ei_c)])
    return packed.reshape(NC * 2 * NS * NG, J, K)


def kernel(x, ei_supplies, ei_competes, Wl0s, bl0s, Wr0s, Wl0c, bl0c, Wr0c,
           Wl1s, bl1s, Wr1s, Wl1c, bl1c, Wr1c, linW, linb):
    ei = _pack_edges(ei_supplies, ei_competes)
    zrow_h = jnp.zeros((K, D), jnp.float32)

    ones_h = jnp.ones((K, D), jnp.float32)
    cnt_pk = _sc_cnt(ei, zrow_h, ones_h)
    cnt = cnt_pk.reshape(NC, NACC, D)[:, :N, 0].T
    acc_a = _sc_agg(ei, x, zrow_h).reshape(NC, NACC, D)
    h = _tc0(acc_a, cnt, x, Wl0s, Wl0c, Wr0s, Wr0c,
             bl0s.reshape(1, D), bl0c.reshape(1, D))
    acc_b = _sc_agg(ei, h, zrow_h).reshape(NC, NACC, D)
    return _tc1(acc_b, cnt, h, Wl1s, Wl1c, Wr1s, Wr1c,
                bl1s.reshape(1, D), bl1c.reshape(1, D),
                linW, linb.reshape(1, D))


# speedup vs baseline: 3.4230x; 1.1692x over previous
import jax
import jax.numpy as jnp
from jax import lax
from jax.experimental import pallas as pl
from jax.experimental.pallas import tpu as pltpu
from jax.experimental.pallas import tpu_sc as plsc

N = 10000
D = 128
E = 320000
NC = 2
NS = 16
K = 128
CB = 4
NB = 40
W = NB * CB * K
EPAD = W * NS
NACC = 10112
RPT = NACC // NS
RB = 1000
ZCH = 64
_CHUNKS = ([(i * ZCH, ZCH) for i in range(RPT // ZCH)]
           + [(RPT - RPT % ZCH, RPT % ZCH)])


def _sc_agg_body(ei, table, zrow_h, acc_out,
                 sidx, didx, rows, zrow, acc, isem, gsem, ssem):
    c = lax.axis_index("c")
    s = lax.axis_index("s")
    base = s * RPT

    pltpu.sync_copy(zrow_h, zrow)
    for off, sz in _CHUNKS:
        pltpu.sync_copy(zrow.at[pl.ds(0, sz)], acc.at[pl.ds(base + off, sz)])
    plsc.subcore_barrier()

    wid = (c * 2 + 0) * NS + s
    wid_d = (c * 2 + 1) * NS + s

    pltpu.async_copy(ei.at[wid * NB], sidx.at[0], isem)
    pltpu.async_copy(ei.at[wid_d * NB], didx.at[0], isem)

    def _block(nb, carry):
        cb = lax.rem(nb, 2)
        pb = lax.rem(nb + 1, 2)
        pltpu.make_async_copy(ei.at[wid * NB + nb], sidx.at[cb], isem).wait()
        pltpu.make_async_copy(ei.at[wid_d * NB + nb], didx.at[cb],
                              isem).wait()

        @pl.when(nb + 1 < NB)
        def _prefetch():
            pltpu.async_copy(ei.at[wid * NB + nb + 1], sidx.at[pb], isem)
            pltpu.async_copy(ei.at[wid_d * NB + nb + 1], didx.at[pb], isem)

        def fire_g(t):
            pltpu.async_copy(table.at[sidx.at[cb, t]], rows.at[t % 2], gsem)

        def wait_g(t):
            pltpu.make_async_copy(table.at[sidx.at[cb, t]], rows.at[t % 2],
                                  gsem).wait()

        def fire_s(t):
            pltpu.async_copy(rows.at[t % 2], acc.at[didx.at[cb, t]], ssem,
                             add=True)

        def wait_s(t):
            pltpu.make_async_copy(rows.at[t % 2], acc.at[didx.at[cb, t]],
                                  ssem).wait()

        fire_g(0)
        fire_g(1)
        wait_g(0)
        fire_s(0)
        wait_g(1)
        fire_s(1)
        wait_s(0)
        fire_g(2)
        wait_s(1)
        fire_g(3)
        wait_g(2)
        fire_s(2)
        wait_g(3)
        fire_s(3)
        wait_s(2)
        wait_s(3)
        return carry

    lax.fori_loop(0, NB, _block, 0)
    plsc.subcore_barrier()

    out_base = c * NACC + base
    for off, sz in _CHUNKS:
        pltpu.sync_copy(acc.at[pl.ds(base + off, sz)], zrow.at[pl.ds(0, sz)])
        pltpu.sync_copy(zrow.at[pl.ds(0, sz)],
                        acc_out.at[pl.ds(out_base + off, sz)])


_sc_agg = pl.kernel(
    _sc_agg_body,
    out_type=jax.ShapeDtypeStruct((NC * NACC, D), jnp.float32),
    mesh=plsc.VectorSubcoreMesh(core_axis_name="c", subcore_axis_name="s",
                                num_cores=NC, num_subcores=NS),
    scratch_types=[
        pltpu.VMEM((2, CB, K), jnp.int32),
        pltpu.VMEM((2, CB, K), jnp.int32),
        pltpu.VMEM((2, K, D), jnp.float32),
        pltpu.VMEM((ZCH, D), jnp.float32),
        pltpu.VMEM_SHARED((NACC, D), jnp.float32),
        pltpu.SemaphoreType.DMA,
        pltpu.SemaphoreType.DMA,
        pltpu.SemaphoreType.DMA,
    ],
)


def _sc_cnt_body(ei, zrow_h, ones_h, cnt_out,
                 didx, onesb, zrow, cnt, isem, ssem):
    c = lax.axis_index("c")
    s = lax.axis_index("s")
    base = s * RPT

    pltpu.sync_copy(zrow_h, zrow)
    pltpu.sync_copy(ones_h, onesb)
    for off, sz in _CHUNKS:
        pltpu.sync_copy(zrow.at[pl.ds(0, sz)], cnt.at[pl.ds(base + off, sz)])
    plsc.subcore_barrier()

    wid_d = (c * 2 + 1) * NS + s
    pltpu.async_copy(ei.at[wid_d * NB], didx.at[0], isem)

    def _block(nb, carry):
        cb = lax.rem(nb, 2)
        pb = lax.rem(nb + 1, 2)
        pltpu.make_async_copy(ei.at[wid_d * NB + nb], didx.at[cb],
                              isem).wait()

        @pl.when(nb + 1 < NB)
        def _prefetch():
            pltpu.async_copy(ei.at[wid_d * NB + nb + 1], didx.at[pb], isem)

        for t in range(CB):
            pltpu.async_copy(onesb, cnt.at[didx.at[cb, t]], ssem, add=True)
        for t in range(CB):
            pltpu.make_async_copy(onesb, cnt.at[didx.at[cb, t]], ssem).wait()
        return carry

    lax.fori_loop(0, NB, _block, 0)
    plsc.subcore_barrier()

    out_base = c * NACC + base
    for off, sz in _CHUNKS:
        pltpu.sync_copy(cnt.at[pl.ds(base + off, sz)], zrow.at[pl.ds(0, sz)])
        pltpu.sync_copy(zrow.at[pl.ds(0, sz)],
                        cnt_out.at[pl.ds(out_base + off, sz)])


_sc_cnt = pl.kernel(
    _sc_cnt_body,
    out_type=jax.ShapeDtypeStruct((NC * NACC, D), jnp.float32),
    mesh=plsc.VectorSubcoreMesh(core_axis_name="c", subcore_axis_name="s",
                                num_cores=NC, num_subcores=NS),
    scratch_types=[
        pltpu.VMEM((2, CB, K), jnp.int32),
        pltpu.VMEM((K, D), jnp.float32),
        pltpu.VMEM((ZCH, D), jnp.float32),
        pltpu.VMEM_SHARED((NACC, D), jnp.float32),
        pltpu.SemaphoreType.DMA,
        pltpu.SemaphoreType.DMA,
    ],
)


def _tc0_body(acc_ref, cnt_ref, x_ref, wls_ref, wlc_ref, wrs_ref, wrc_ref,
              bls_ref, blc_ref, h_ref):
    inv_s = 1.0 / jnp.maximum(cnt_ref[:, 0:1], 1.0)
    inv_c = 1.0 / jnp.maximum(cnt_ref[:, 1:2], 1.0)
    r = jnp.dot(acc_ref[0] * inv_s, wls_ref[...],
                preferred_element_type=jnp.float32)
    r = r + jnp.dot(acc_ref[1] * inv_c, wlc_ref[...],
                    preferred_element_type=jnp.float32)
    r = r + jnp.dot(x_ref[...], wrs_ref[...] + wrc_ref[...],
                    preferred_element_type=jnp.float32)
    r = r + bls_ref[...] + blc_ref[...]
    h_ref[...] = jnp.maximum(r, 0.0)


def _tc1_body(acc_ref, cnt_ref, h_ref, wls_ref, wlc_ref, wrs_ref, wrc_ref,
              bls_ref, blc_ref, lw_ref, lb_ref, o_ref):
    inv_s = 1.0 / jnp.maximum(cnt_ref[:, 0:1], 1.0)
    inv_c = 1.0 / jnp.maximum(cnt_ref[:, 1:2], 1.0)
    g = jnp.dot(acc_ref[0] * inv_s, wls_ref[...],
                preferred_element_type=jnp.float32)
    g = g + jnp.dot(acc_ref[1] * inv_c, wlc_ref[...],
                    preferred_element_type=jnp.float32)
    g = g + jnp.dot(h_ref[...], wrs_ref[...] + wrc_ref[...],
                    preferred_element_type=jnp.float32)
    g = jnp.maximum(g + bls_ref[...] + blc_ref[...], 0.0)
    o_ref[...] = jnp.dot(g, lw_ref[...],
                         preferred_element_type=jnp.float32) + lb_ref[...]


def _tc0(acc, cnt, x, wls, wlc, wrs, wrc, bls, blc):
    wspec = pl.BlockSpec((D, D), lambda i: (0, 0))
    bspec = pl.BlockSpec((1, D), lambda i: (0, 0))
    return pl.pallas_call(
        _tc0_body,
        grid=(N // RB,),
        in_specs=[
            pl.BlockSpec((NC, RB, D), lambda i: (0, i, 0)),
            pl.BlockSpec((RB, NC), lambda i: (i, 0)),
            pl.BlockSpec((RB, D), lambda i: (i, 0)),
            wspec, wspec, wspec, wspec, bspec, bspec,
        ],
        out_specs=pl.BlockSpec((RB, D), lambda i: (i, 0)),
        out_shape=jax.ShapeDtypeStruct((N, D), jnp.float32),
    )(acc, cnt, x, wls, wlc, wrs, wrc, bls, blc)


def _tc1(acc, cnt, h, wls, wlc, wrs, wrc, bls, blc, lw, lb):
    wspec = pl.BlockSpec((D, D), lambda i: (0, 0))
    bspec = pl.BlockSpec((1, D), lambda i: (0, 0))
    return pl.pallas_call(
        _tc1_body,
        grid=(N // RB,),
        in_specs=[
            pl.BlockSpec((NC, RB, D), lambda i: (0, i, 0)),
            pl.BlockSpec((RB, NC), lambda i: (i, 0)),
            pl.BlockSpec((RB, D), lambda i: (i, 0)),
            wspec, wspec, wspec, wspec, bspec, bspec, wspec, bspec,
        ],
        out_specs=pl.BlockSpec((RB, D), lambda i: (i, 0)),
        out_shape=jax.ShapeDtypeStruct((N, D), jnp.float32),
    )(acc, cnt, h, wls, wlc, wrs, wrc, bls, blc, lw, lb)


def _pack_edges(ei_s, ei_c):
    pad = EPAD - E

    def prep(ei):
        src = jnp.concatenate([ei[0], jnp.zeros((pad,), jnp.int32)])
        dst = jnp.concatenate([ei[1], jnp.full((pad,), N, jnp.int32)])
        return jnp.stack([src, dst])

    packed = jnp.stack([prep(ei_s), prep(ei_c)])
    return packed.reshape(NC * 2 * NS * NB, CB, K)


def kernel(x, ei_supplies, ei_competes, Wl0s, bl0s, Wr0s, Wl0c, bl0c, Wr0c,
           Wl1s, bl1s, Wr1s, Wl1c, bl1c, Wr1c, linW, linb):
    ei = _pack_edges(ei_supplies, ei_competes)
    zrow_h = jnp.zeros((ZCH, D), jnp.float32)

    ones_h = jnp.ones((K, D), jnp.float32)
    cnt_pk = _sc_cnt(ei, zrow_h, ones_h)
    cnt = cnt_pk.reshape(NC, NACC, D)[:, :N, 0].T
    acc_a = _sc_agg(ei, x, zrow_h).reshape(NC, NACC, D)
    h = _tc0(acc_a, cnt, x, Wl0s, Wl0c, Wr0s, Wr0c,
             bl0s.reshape(1, D), bl0c.reshape(1, D))
    acc_b = _sc_agg(ei, h, zrow_h).reshape(NC, NACC, D)
    return _tc1(acc_b, cnt, h, Wl1s, Wl1c, Wr1s, Wr1c,
                bl1s.reshape(1, D), bl1c.reshape(1, D),
                linW, linb.reshape(1, D))


# speedup vs baseline: 3.4945x; 1.0209x over previous
import jax
import jax.numpy as jnp
from jax import lax
from jax.experimental import pallas as pl
from jax.experimental.pallas import tpu as pltpu
from jax.experimental.pallas import tpu_sc as plsc

N = 10000
D = 128
E = 320000
NC = 2
NS = 16
K = 64
CB = 8
NB = 40
NBUF = 4
W = NB * CB * K
EPAD = W * NS
NACC = 10112
RPT = NACC // NS
RB = 1000
ZCH = 64
_CHUNKS = ([(i * ZCH, ZCH) for i in range(RPT // ZCH)]
           + [(RPT - RPT % ZCH, RPT % ZCH)])


def _sc_agg_body(ei, table, zrow_h, acc_out,
                 sidx, didx, rows, zrow, acc, isem, gsem, ssem):
    c = lax.axis_index("c")
    s = lax.axis_index("s")
    base = s * RPT

    pltpu.sync_copy(zrow_h, zrow)
    for off, sz in _CHUNKS:
        pltpu.sync_copy(zrow.at[pl.ds(0, sz)], acc.at[pl.ds(base + off, sz)])
    plsc.subcore_barrier()

    wid = (c * 2 + 0) * NS + s
    wid_d = (c * 2 + 1) * NS + s

    pltpu.async_copy(ei.at[wid * NB], sidx.at[0], isem)
    pltpu.async_copy(ei.at[wid_d * NB], didx.at[0], isem)

    def _block(nb, carry):
        cb = lax.rem(nb, 2)
        pb = lax.rem(nb + 1, 2)
        pltpu.make_async_copy(ei.at[wid * NB + nb], sidx.at[cb], isem).wait()
        pltpu.make_async_copy(ei.at[wid_d * NB + nb], didx.at[cb],
                              isem).wait()

        @pl.when(nb + 1 < NB)
        def _prefetch():
            pltpu.async_copy(ei.at[wid * NB + nb + 1], sidx.at[pb], isem)
            pltpu.async_copy(ei.at[wid_d * NB + nb + 1], didx.at[pb], isem)

        def fire_g(t):
            pltpu.async_copy(table.at[sidx.at[cb, t]], rows.at[t % NBUF],
                             gsem)

        def wait_g(t):
            pltpu.make_async_copy(table.at[sidx.at[cb, t]],
                                  rows.at[t % NBUF], gsem).wait()

        def fire_s(t):
            pltpu.async_copy(rows.at[t % NBUF], acc.at[didx.at[cb, t]], ssem,
                             add=True)

        def wait_s(t):
            pltpu.make_async_copy(rows.at[t % NBUF], acc.at[didx.at[cb, t]],
                                  ssem).wait()

        for t in range(NBUF - 1):
            fire_g(t)
        for t in range(CB):
            wait_g(t)
            fire_s(t)
            nxt = t + NBUF - 1
            if nxt < CB:
                if t >= 1:
                    wait_s(t - 1)
                fire_g(nxt)
        for t in range(CB - NBUF, CB):
            wait_s(t)
        return carry

    lax.fori_loop(0, NB, _block, 0)
    plsc.subcore_barrier()

    out_base = c * NACC + base
    for off, sz in _CHUNKS:
        pltpu.sync_copy(acc.at[pl.ds(base + off, sz)], zrow.at[pl.ds(0, sz)])
        pltpu.sync_copy(zrow.at[pl.ds(0, sz)],
                        acc_out.at[pl.ds(out_base + off, sz)])


_sc_agg = pl.kernel(
    _sc_agg_body,
    out_type=jax.ShapeDtypeStruct((NC * NACC, D), jnp.float32),
    mesh=plsc.VectorSubcoreMesh(core_axis_name="c", subcore_axis_name="s",
                                num_cores=NC, num_subcores=NS),
    scratch_types=[
        pltpu.VMEM((2, CB, K), jnp.int32),
        pltpu.VMEM((2, CB, K), jnp.int32),
        pltpu.VMEM((NBUF, K, D), jnp.float32),
        pltpu.VMEM((ZCH, D), jnp.float32),
        pltpu.VMEM_SHARED((NACC, D), jnp.float32),
        pltpu.SemaphoreType.DMA,
        pltpu.SemaphoreType.DMA,
        pltpu.SemaphoreType.DMA,
    ],
)


def _sc_cnt_body(ei, zrow_h, ones_h, cnt_out,
                 didx, onesb, zrow, cnt, isem, ssem):
    c = lax.axis_index("c")
    s = lax.axis_index("s")
    base = s * RPT

    pltpu.sync_copy(zrow_h, zrow)
    pltpu.sync_copy(ones_h, onesb)
    for off, sz in _CHUNKS:
        pltpu.sync_copy(zrow.at[pl.ds(0, sz)], cnt.at[pl.ds(base + off, sz)])
    plsc.subcore_barrier()

    wid_d = (c * 2 + 1) * NS + s
    pltpu.async_copy(ei.at[wid_d * NB], didx.at[0], isem)

    def _block(nb, carry):
        cb = lax.rem(nb, 2)
        pb = lax.rem(nb + 1, 2)
        pltpu.make_async_copy(ei.at[wid_d * NB + nb], didx.at[cb],
                              isem).wait()

        @pl.when(nb + 1 < NB)
        def _prefetch():
            pltpu.async_copy(ei.at[wid_d * NB + nb + 1], didx.at[pb], isem)

        for t in range(CB):
            pltpu.async_copy(onesb, cnt.at[didx.at[cb, t]], ssem, add=True)
        for t in range(CB):
            pltpu.make_async_copy(onesb, cnt.at[didx.at[cb, t]], ssem).wait()
        return carry

    lax.fori_loop(0, NB, _block, 0)
    plsc.subcore_barrier()

    out_base = c * NACC + base
    for off, sz in _CHUNKS:
        pltpu.sync_copy(cnt.at[pl.ds(base + off, sz)], zrow.at[pl.ds(0, sz)])
        pltpu.sync_copy(zrow.at[pl.ds(0, sz)],
                        cnt_out.at[pl.ds(out_base + off, sz)])


_sc_cnt = pl.kernel(
    _sc_cnt_body,
    out_type=jax.ShapeDtypeStruct((NC * NACC, D), jnp.float32),
    mesh=plsc.VectorSubcoreMesh(core_axis_name="c", subcore_axis_name="s",
                                num_cores=NC, num_subcores=NS),
    scratch_types=[
        pltpu.VMEM((2, CB, K), jnp.int32),
        pltpu.VMEM((K, D), jnp.float32),
        pltpu.VMEM((ZCH, D), jnp.float32),
        pltpu.VMEM_SHARED((NACC, D), jnp.float32),
        pltpu.SemaphoreType.DMA,
        pltpu.SemaphoreType.DMA,
    ],
)


def _tc0_body(acc_ref, cnt_ref, x_ref, wls_ref, wlc_ref, wrs_ref, wrc_ref,
              bls_ref, blc_ref, h_ref):
    inv_s = 1.0 / jnp.maximum(cnt_ref[:, 0:1], 1.0)
    inv_c = 1.0 / jnp.maximum(cnt_ref[:, 1:2], 1.0)
    r = jnp.dot(acc_ref[0] * inv_s, wls_ref[...],
                preferred_element_type=jnp.float32)
    r = r + jnp.dot(acc_ref[1] * inv_c, wlc_ref[...],
                    preferred_element_type=jnp.float32)
    r = r + jnp.dot(x_ref[...], wrs_ref[...] + wrc_ref[...],
                    preferred_element_type=jnp.float32)
    r = r + bls_ref[...] + blc_ref[...]
    h_ref[...] = jnp.maximum(r, 0.0)


def _tc1_body(acc_ref, cnt_ref, h_ref, wls_ref, wlc_ref, wrs_ref, wrc_ref,
              bls_ref, blc_ref, lw_ref, lb_ref, o_ref):
    inv_s = 1.0 / jnp.maximum(cnt_ref[:, 0:1], 1.0)
    inv_c = 1.0 / jnp.maximum(cnt_ref[:, 1:2], 1.0)
    g = jnp.dot(acc_ref[0] * inv_s, wls_ref[...],
                preferred_element_type=jnp.float32)
    g = g + jnp.dot(acc_ref[1] * inv_c, wlc_ref[...],
                    preferred_element_type=jnp.float32)
    g = g + jnp.dot(h_ref[...], wrs_ref[...] + wrc_ref[...],
                    preferred_element_type=jnp.float32)
    g = jnp.maximum(g + bls_ref[...] + blc_ref[...], 0.0)
    o_ref[...] = jnp.dot(g, lw_ref[...],
                         preferred_element_type=jnp.float32) + lb_ref[...]


def _tc0(acc, cnt, x, wls, wlc, wrs, wrc, bls, blc):
    wspec = pl.BlockSpec((D, D), lambda i: (0, 0))
    bspec = pl.BlockSpec((1, D), lambda i: (0, 0))
    return pl.pallas_call(
        _tc0_body,
        grid=(N // RB,),
        in_specs=[
            pl.BlockSpec((NC, RB, D), lambda i: (0, i, 0)),
            pl.BlockSpec((RB, NC), lambda i: (i, 0)),
            pl.BlockSpec((RB, D), lambda i: (i, 0)),
            wspec, wspec, wspec, wspec, bspec, bspec,
        ],
        out_specs=pl.BlockSpec((RB, D), lambda i: (i, 0)),
        out_shape=jax.ShapeDtypeStruct((N, D), jnp.float32),
    )(acc, cnt, x, wls, wlc, wrs, wrc, bls, blc)


def _tc1(acc, cnt, h, wls, wlc, wrs, wrc, bls, blc, lw, lb):
    wspec = pl.BlockSpec((D, D), lambda i: (0, 0))
    bspec = pl.BlockSpec((1, D), lambda i: (0, 0))
    return pl.pallas_call(
        _tc1_body,
        grid=(N // RB,),
        in_specs=[
            pl.BlockSpec((NC, RB, D), lambda i: (0, i, 0)),
            pl.BlockSpec((RB, NC), lambda i: (i, 0)),
            pl.BlockSpec((RB, D), lambda i: (i, 0)),
            wspec, wspec, wspec, wspec, bspec, bspec, wspec, bspec,
        ],
        out_specs=pl.BlockSpec((RB, D), lambda i: (i, 0)),
        out_shape=jax.ShapeDtypeStruct((N, D), jnp.float32),
    )(acc, cnt, h, wls, wlc, wrs, wrc, bls, blc, lw, lb)


def _pack_edges(ei_s, ei_c):
    pad = EPAD - E

    def prep(ei):
        src = jnp.concatenate([ei[0], jnp.zeros((pad,), jnp.int32)])
        dst = jnp.concatenate([ei[1], jnp.full((pad,), N, jnp.int32)])
        return jnp.stack([src, dst])

    packed = jnp.stack([prep(ei_s), prep(ei_c)])
    return packed.reshape(NC * 2 * NS * NB, CB, K)


def kernel(x, ei_supplies, ei_competes, Wl0s, bl0s, Wr0s, Wl0c, bl0c, Wr0c,
           Wl1s, bl1s, Wr1s, Wl1c, bl1c, Wr1c, linW, linb):
    ei = _pack_edges(ei_supplies, ei_competes)
    zrow_h = jnp.zeros((ZCH, D), jnp.float32)

    ones_h = jnp.ones((K, D), jnp.float32)
    cnt_pk = _sc_cnt(ei, zrow_h, ones_h)
    cnt = cnt_pk.reshape(NC, NACC, D)[:, :N, 0].T
    acc_a = _sc_agg(ei, x, zrow_h).reshape(NC, NACC, D)
    h = _tc0(acc_a, cnt, x, Wl0s, Wl0c, Wr0s, Wr0c,
             bl0s.reshape(1, D), bl0c.reshape(1, D))
    acc_b = _sc_agg(ei, h, zrow_h).reshape(NC, NACC, D)
    return _tc1(acc_b, cnt, h, Wl1s, Wl1c, Wr1s, Wr1c,
                bl1s.reshape(1, D), bl1c.reshape(1, D),
                linW, linb.reshape(1, D))
